# baseline (speedup 1.0000x reference)
import functools

import jax
import jax.numpy as jnp
from jax import lax
from jax.experimental import pallas as pl
from jax.experimental.pallas import tpu as pltpu
from jax.experimental.pallas import tpu_sc as plsc

N_NODES = 100000
IN_DIM = 16
NON_ZERO = 128
FEATURES_DIM = 200
ZERO_ROWS = FEATURES_DIM - NON_ZERO
NUM_CORES = 2
NUM_SUBCORES = 16
NUM_WORKERS = NUM_CORES * NUM_SUBCORES
LANES = 16

CW = 256
NUM_FULL = N_NODES // CW
VPC = CW // LANES
TAIL1_OFF = NUM_FULL * CW
TAIL1_W = 128
TAIL2_OFF = TAIL1_OFF + TAIL1_W
TAIL2_W = N_NODES - TAIL2_OFF
TAIL1_WORKER = 28
TAIL2_WORKER = 30
MAX_PAIRS = ((NUM_FULL + NUM_WORKERS - 1) // NUM_WORKERS + 1) // 2

_SRC = [ll * ll + m
        for ll in range(4) for _s in range(8) for m in range(2 * ll + 1)]
assert len(_SRC) == NON_ZERO

_GATHER_DNUMS = lax.GatherDimensionNumbers(
    offset_dims=(), collapsed_slice_dims=(0,), start_index_map=(0,))


def _splat(vec, lane):
    idx = jnp.full((LANES,), lane, jnp.int32)
    return lax.gather(vec, idx[:, None], _GATHER_DNUMS, (1,),
                      mode=lax.GatherScatterMode.PROMISE_IN_BOUNDS)


def _sc_kernel_body(cd_hbm, ovl_hbm, out_hbm,
                    in0, in1, do0, do1, zb, in_r, do_r, zb_r, ovl_v,
                    sem_in0, sem_in1, sem_out0, sem_out1):
    wid = lax.axis_index("s") * NUM_CORES + lax.axis_index("c")

    pltpu.sync_copy(ovl_hbm, ovl_v)
    ovl_vecs = [ovl_v[pl.ds(LANES * g, LANES)] for g in range(NON_ZERO // LANES)]

    in_bufs = (in0, in1)
    do_bufs = (do0, do1)
    sem_ins = (sem_in0, sem_in1)
    sem_outs = (sem_out0, sem_out1)

    zeros = jnp.zeros((LANES,), jnp.float32)

    def zero_row(r, carry):
        for v in range(VPC):
            zb[r, pl.ds(LANES * v, LANES)] = zeros
        for v in range(TAIL2_W // LANES):
            zb_r[r, pl.ds(LANES * v, LANES)] = zeros
        return carry

    lax.fori_loop(0, ZERO_ROWS, zero_row, 0)

    def compute_cols(in_ref, out_ref, n_vregs):
        def col_body(v, carry):
            off = pl.ds(LANES * v, LANES)
            for src in range(IN_DIM):
                row = in_ref[src, off]
                for j, s in enumerate(_SRC):
                    if s != src:
                        continue
                    scale = _splat(ovl_vecs[j // LANES], j % LANES)
                    out_ref[j, off] = row * scale
            return carry

        lax.fori_loop(0, n_vregs, col_body, 0)

    @pl.when(wid == TAIL1_WORKER)
    def _():
        cols = pl.ds(TAIL1_OFF, TAIL1_W)
        half = pl.ds(0, TAIL1_W)
        cp = pltpu.make_async_copy(
            cd_hbm.at[:, cols], in0.at[:, half], sem_in0)
        cp.start()
        cp.wait()
        compute_cols(in0, do0, TAIL1_W // LANES)
        cpd = pltpu.make_async_copy(
            do0.at[:, half], out_hbm.at[pl.ds(0, NON_ZERO), cols], sem_out0)
        cpz = pltpu.make_async_copy(
            zb.at[:, half], out_hbm.at[pl.ds(NON_ZERO, ZERO_ROWS), cols],
            sem_out1)
        cpd.start()
        cpz.start()
        cpd.wait()
        cpz.wait()

    @pl.when(wid == TAIL2_WORKER)
    def _():
        cols = pl.ds(TAIL2_OFF, TAIL2_W)
        cp = pltpu.make_async_copy(cd_hbm.at[:, cols], in_r, sem_in0)
        cp.start()
        cp.wait()
        compute_cols(in_r, do_r, TAIL2_W // LANES)
        cpd = pltpu.make_async_copy(
            do_r, out_hbm.at[pl.ds(0, NON_ZERO), cols], sem_out0)
        cpz = pltpu.make_async_copy(
            zb_r, out_hbm.at[pl.ds(NON_ZERO, ZERO_ROWS), cols], sem_out1)
        cpd.start()
        cpz.start()
        cpd.wait()
        cpz.wait()

    my_chunks = (NUM_FULL - 1 - wid) // NUM_WORKERS + 1

    def col_of(j):
        return pl.multiple_of((wid + j * NUM_WORKERS) * CW, CW)

    def in_copy(j, b):
        return pltpu.make_async_copy(
            cd_hbm.at[:, pl.ds(col_of(j), CW)], in_bufs[b], sem_ins[b])

    def out_data(j, b):
        return pltpu.make_async_copy(
            do_bufs[b],
            out_hbm.at[pl.ds(0, NON_ZERO), pl.ds(col_of(j), CW)],
            sem_outs[b])

    def out_zero(j, b):
        return pltpu.make_async_copy(
            zb,
            out_hbm.at[pl.ds(NON_ZERO, ZERO_ROWS), pl.ds(col_of(j), CW)],
            sem_outs[b])

    in_copy(0, 0).start()

    def pair_body(i, carry):
        for b in range(2):
            j = i * 2 + b

            @pl.when(j < my_chunks)
            def _():
                @pl.when(j + 1 < my_chunks)
                def _():
                    in_copy(j + 1, 1 - b).start()

                in_copy(j, b).wait()

                @pl.when(j >= 2)
                def _():
                    out_data(j, b).wait()
                    out_zero(j, b).wait()

                compute_cols(in_bufs[b], do_bufs[b], VPC)
                out_data(j, b).start()
                out_zero(j, b).start()
        return carry

    lax.fori_loop(0, MAX_PAIRS, pair_body, 0)

    for b in range(2):
        @pl.when(my_chunks >= b + 1)
        def _():
            out_data(b, b).wait()
            out_zero(b, b).wait()


def kernel(charge_density, overlap_constants, select_indices):
    del select_indices
    cd_t = charge_density.T
    mesh = plsc.VectorSubcoreMesh(core_axis_name="c", subcore_axis_name="s")
    run = functools.partial(
        pl.kernel,
        mesh=mesh,
        out_type=jax.ShapeDtypeStruct((FEATURES_DIM, N_NODES), jnp.float32),
        scratch_types=[
            pltpu.VMEM((IN_DIM, CW), jnp.float32),
            pltpu.VMEM((IN_DIM, CW), jnp.float32),
            pltpu.VMEM((NON_ZERO, CW), jnp.float32),
            pltpu.VMEM((NON_ZERO, CW), jnp.float32),
            pltpu.VMEM((ZERO_ROWS, CW), jnp.float32),
            pltpu.VMEM((IN_DIM, TAIL2_W), jnp.float32),
            pltpu.VMEM((NON_ZERO, TAIL2_W), jnp.float32),
            pltpu.VMEM((ZERO_ROWS, TAIL2_W), jnp.float32),
            pltpu.VMEM((NON_ZERO,), jnp.float32),
            pltpu.SemaphoreType.DMA,
            pltpu.SemaphoreType.DMA,
            pltpu.SemaphoreType.DMA,
            pltpu.SemaphoreType.DMA,
        ],
    )(_sc_kernel_body)
    out_t = run(cd_t, overlap_constants)
    return out_t.T


# speedup vs baseline: 1.1202x; 1.1202x over previous
import functools

import jax
import jax.numpy as jnp
from jax import lax
from jax.experimental import pallas as pl
from jax.experimental.pallas import tpu as pltpu
from jax.experimental.pallas import tpu_sc as plsc

N_NODES = 100000
IN_DIM = 16
NON_ZERO = 128
FEATURES_DIM = 200
NUM_CORES = 2
NUM_SUBCORES = 16
NUM_WORKERS = NUM_CORES * NUM_SUBCORES
LANES = 16
NBUF = 3

CW = 128
NUM_FULL = N_NODES // CW
REM = N_NODES - NUM_FULL * CW
REM_OFF = NUM_FULL * CW
REM_WORKER = 30
VPC = CW // LANES
VPC_REM = REM // LANES
MAX_TRIPLES = ((NUM_FULL + NUM_WORKERS - 1) // NUM_WORKERS + NBUF - 1) // NBUF

_SRC = [ll * ll + m
        for ll in range(4) for _s in range(8) for m in range(2 * ll + 1)]
assert len(_SRC) == NON_ZERO

_GATHER_DNUMS = lax.GatherDimensionNumbers(
    offset_dims=(), collapsed_slice_dims=(0,), start_index_map=(0,))


def _splat(vec, lane):
    idx = jnp.full((LANES,), lane, jnp.int32)
    return lax.gather(vec, idx[:, None], _GATHER_DNUMS, (1,),
                      mode=lax.GatherScatterMode.PROMISE_IN_BOUNDS)


def _sc_kernel_body(cd_hbm, ovl_hbm, out_hbm,
                    in0, in1, in2, out0, out1, out2, in_r, out_r, ovl_v,
                    sem_in0, sem_in1, sem_in2,
                    sem_out0, sem_out1, sem_out2):
    wid = lax.axis_index("s") * NUM_CORES + lax.axis_index("c")

    pltpu.sync_copy(ovl_hbm, ovl_v)
    ovl_vecs = [ovl_v[pl.ds(LANES * g, LANES)] for g in range(NON_ZERO // LANES)]

    in_bufs = (in0, in1, in2)
    out_bufs = (out0, out1, out2)
    sem_ins = (sem_in0, sem_in1, sem_in2)
    sem_outs = (sem_out0, sem_out1, sem_out2)

    zeros = jnp.zeros((LANES,), jnp.float32)

    def zero_row(r, carry):
        for ob in out_bufs:
            for v in range(VPC):
                ob[NON_ZERO + r, pl.ds(LANES * v, LANES)] = zeros
        for v in range(VPC_REM):
            out_r[NON_ZERO + r, pl.ds(LANES * v, LANES)] = zeros
        return carry

    lax.fori_loop(0, FEATURES_DIM - NON_ZERO, zero_row, 0)

    def compute_cols(in_ref, out_ref, n_vregs):
        def col_body(v, carry):
            off = pl.ds(LANES * v, LANES)
            for src in range(IN_DIM):
                row = in_ref[src, off]
                for j, s in enumerate(_SRC):
                    if s != src:
                        continue
                    scale = _splat(ovl_vecs[j // LANES], j % LANES)
                    out_ref[j, off] = row * scale
            return carry

        lax.fori_loop(0, n_vregs, col_body, 0)

    @pl.when(wid == REM_WORKER)
    def _():
        rem = pl.ds(REM_OFF, REM)
        cp = pltpu.make_async_copy(cd_hbm.at[:, rem], in_r, sem_in0)
        cp.start()
        cp.wait()
        compute_cols(in_r, out_r, VPC_REM)
        cp2 = pltpu.make_async_copy(out_r, out_hbm.at[:, rem], sem_out0)
        cp2.start()
        cp2.wait()

    my_chunks = (NUM_FULL - 1 - wid) // NUM_WORKERS + 1

    def col_of(j):
        return pl.multiple_of((wid + j * NUM_WORKERS) * CW, CW)

    def in_copy(j, b):
        return pltpu.make_async_copy(
            cd_hbm.at[:, pl.ds(col_of(j), CW)], in_bufs[b], sem_ins[b])

    def out_copy(j, b):
        return pltpu.make_async_copy(
            out_bufs[b], out_hbm.at[:, pl.ds(col_of(j), CW)], sem_outs[b])

    in_copy(0, 0).start()

    def triple_body(i, carry):
        for b in range(NBUF):
            j = i * NBUF + b

            @pl.when(j < my_chunks)
            def _():
                @pl.when(j + 1 < my_chunks)
                def _():
                    in_copy(j + 1, (b + 1) % NBUF).start()

                in_copy(j, b).wait()

                @pl.when(j >= NBUF)
                def _():
                    out_copy(j, b).wait()

                compute_cols(in_bufs[b], out_bufs[b], VPC)
                out_copy(j, b).start()
        return carry

    lax.fori_loop(0, MAX_TRIPLES, triple_body, 0)

    for b in range(NBUF):
        @pl.when(my_chunks >= b + 1)
        def _():
            out_copy(b, b).wait()


def kernel(charge_density, overlap_constants, select_indices):
    del select_indices
    cd_t = charge_density.T
    mesh = plsc.VectorSubcoreMesh(core_axis_name="c", subcore_axis_name="s")
    run = functools.partial(
        pl.kernel,
        mesh=mesh,
        out_type=jax.ShapeDtypeStruct((FEATURES_DIM, N_NODES), jnp.float32),
        scratch_types=[
            pltpu.VMEM((IN_DIM, CW), jnp.float32),
            pltpu.VMEM((IN_DIM, CW), jnp.float32),
            pltpu.VMEM((IN_DIM, CW), jnp.float32),
            pltpu.VMEM((FEATURES_DIM, CW), jnp.float32),
            pltpu.VMEM((FEATURES_DIM, CW), jnp.float32),
            pltpu.VMEM((FEATURES_DIM, CW), jnp.float32),
            pltpu.VMEM((IN_DIM, REM), jnp.float32),
            pltpu.VMEM((FEATURES_DIM, REM), jnp.float32),
            pltpu.VMEM((NON_ZERO,), jnp.float32),
            pltpu.SemaphoreType.DMA,
            pltpu.SemaphoreType.DMA,
            pltpu.SemaphoreType.DMA,
            pltpu.SemaphoreType.DMA,
            pltpu.SemaphoreType.DMA,
            pltpu.SemaphoreType.DMA,
        ],
    )(_sc_kernel_body)
    out_t = run(cd_t, overlap_constants)
    return out_t.T


# speedup vs baseline: 1.2275x; 1.0957x over previous
import functools

import jax
import jax.numpy as jnp
from jax import lax
from jax.experimental import pallas as pl
from jax.experimental.pallas import tpu as pltpu
from jax.experimental.pallas import tpu_sc as plsc

N_NODES = 100000
IN_DIM = 16
NON_ZERO = 128
FEATURES_DIM = 200
NUM_CORES = 2
NUM_SUBCORES = 16
NUM_WORKERS = NUM_CORES * NUM_SUBCORES
LANES = 16

CW = 128
NUM_FULL = N_NODES // CW
REM = N_NODES - NUM_FULL * CW
REM_OFF = NUM_FULL * CW
REM_WORKER = 30
VPC = CW // LANES
VPC_REM = REM // LANES
MAX_PAIRS = ((NUM_FULL + NUM_WORKERS - 1) // NUM_WORKERS + 1) // 2

_SRC = [ll * ll + m
        for ll in range(4) for _s in range(8) for m in range(2 * ll + 1)]
assert len(_SRC) == NON_ZERO

_GATHER_DNUMS = lax.GatherDimensionNumbers(
    offset_dims=(), collapsed_slice_dims=(0,), start_index_map=(0,))


def _splat(vec, lane):
    idx = jnp.full((LANES,), lane, jnp.int32)
    return lax.gather(vec, idx[:, None], _GATHER_DNUMS, (1,),
                      mode=lax.GatherScatterMode.PROMISE_IN_BOUNDS)


def _sc_kernel_body(cd_hbm, ovl_hbm, out_hbm,
                    in0, in1, out0, out1, in_r, out_r, ovl_v,
                    sem_in0, sem_in1, sem_out0, sem_out1):
    wid = lax.axis_index("s") * NUM_CORES + lax.axis_index("c")

    pltpu.sync_copy(ovl_hbm, ovl_v)
    ovl_vecs = [ovl_v[pl.ds(LANES * g, LANES)] for g in range(NON_ZERO // LANES)]

    in_bufs = (in0, in1)
    out_bufs = (out0, out1)
    sem_ins = (sem_in0, sem_in1)
    sem_outs = (sem_out0, sem_out1)

    zeros = jnp.zeros((LANES,), jnp.float32)

    def zero_row(r, carry):
        for ob in out_bufs:
            for v in range(VPC):
                ob[NON_ZERO + r, pl.ds(LANES * v, LANES)] = zeros
        for v in range(VPC_REM):
            out_r[NON_ZERO + r, pl.ds(LANES * v, LANES)] = zeros
        return carry

    lax.fori_loop(0, FEATURES_DIM - NON_ZERO, zero_row, 0)

    def compute_cols(in_ref, out_ref, n_vregs):
        def col_body(v, carry):
            off = pl.ds(LANES * v, LANES)
            for src in range(IN_DIM):
                row = in_ref[src, off]
                for j, s in enumerate(_SRC):
                    if s != src:
                        continue
                    scale = _splat(ovl_vecs[j // LANES], j % LANES)
                    out_ref[j, off] = row * scale
            return carry

        lax.fori_loop(0, n_vregs, col_body, 0)

    @pl.when(wid == REM_WORKER)
    def _():
        rem = pl.ds(REM_OFF, REM)
        cp = pltpu.make_async_copy(cd_hbm.at[:, rem], in_r, sem_in0)
        cp.start()
        cp.wait()
        compute_cols(in_r, out_r, VPC_REM)
        cp2 = pltpu.make_async_copy(out_r, out_hbm.at[:, rem], sem_out0)
        cp2.start()
        cp2.wait()

    my_chunks = (NUM_FULL - 1 - wid) // NUM_WORKERS + 1

    def col_of(j):
        return pl.multiple_of((wid + j * NUM_WORKERS) * CW, CW)

    def in_copy(j, b):
        return pltpu.make_async_copy(
            cd_hbm.at[:, pl.ds(col_of(j), CW)], in_bufs[b], sem_ins[b])

    def out_copy(j, b):
        return pltpu.make_async_copy(
            out_bufs[b], out_hbm.at[:, pl.ds(col_of(j), CW)], sem_outs[b])

    in_copy(0, 0).start()

    def pair_body(i, carry):
        for b in range(2):
            j = i * 2 + b

            @pl.when(j < my_chunks)
            def _():
                @pl.when(j + 1 < my_chunks)
                def _():
                    in_copy(j + 1, 1 - b).start()

                in_copy(j, b).wait()

                @pl.when(j >= 2)
                def _():
                    out_copy(j, b).wait()

                compute_cols(in_bufs[b], out_bufs[b], VPC)
                out_copy(j, b).start()
        return carry

    lax.fori_loop(0, MAX_PAIRS, pair_body, 0)

    for b in range(2):
        @pl.when(my_chunks >= b + 1)
        def _():
            out_copy(b, b).wait()


def kernel(charge_density, overlap_constants, select_indices):
    del select_indices
    cd_t = charge_density.T
    mesh = plsc.VectorSubcoreMesh(core_axis_name="c", subcore_axis_name="s")
    run = functools.partial(
        pl.kernel,
        mesh=mesh,
        out_type=jax.ShapeDtypeStruct((FEATURES_DIM, N_NODES), jnp.float32),
        scratch_types=[
            pltpu.VMEM((IN_DIM, CW), jnp.float32),
            pltpu.VMEM((IN_DIM, CW), jnp.float32),
            pltpu.VMEM((FEATURES_DIM, CW), jnp.float32),
            pltpu.VMEM((FEATURES_DIM, CW), jnp.float32),
            pltpu.VMEM((IN_DIM, REM), jnp.float32),
            pltpu.VMEM((FEATURES_DIM, REM), jnp.float32),
            pltpu.VMEM((NON_ZERO,), jnp.float32),
            pltpu.SemaphoreType.DMA,
            pltpu.SemaphoreType.DMA,
            pltpu.SemaphoreType.DMA,
            pltpu.SemaphoreType.DMA,
        ],
    )(_sc_kernel_body)
    out_t = run(cd_t, overlap_constants)
    return out_t.T


# speedup vs baseline: 1.2372x; 1.0079x over previous
import functools

import jax
import jax.numpy as jnp
from jax import lax
from jax.experimental import pallas as pl
from jax.experimental.pallas import tpu as pltpu
from jax.experimental.pallas import tpu_sc as plsc

N_NODES = 100000
IN_DIM = 16
NON_ZERO = 128
FEATURES_DIM = 200
NUM_CORES = 2
NUM_SUBCORES = 16
NUM_WORKERS = NUM_CORES * NUM_SUBCORES
LANES = 16

CW = 128
NUM_FULL = N_NODES // CW
REM = N_NODES - NUM_FULL * CW
REM_OFF = NUM_FULL * CW
REM_WORKER = 30
VPC = CW // LANES
VPC_REM = REM // LANES
MAX_PAIRS = ((NUM_FULL + NUM_WORKERS - 1) // NUM_WORKERS + 1) // 2

_SRC = [ll * ll + m
        for ll in range(4) for _s in range(8) for m in range(2 * ll + 1)]
assert len(_SRC) == NON_ZERO

_GATHER_DNUMS = lax.GatherDimensionNumbers(
    offset_dims=(), collapsed_slice_dims=(0,), start_index_map=(0,))


def _splat(vec, lane):
    idx = jnp.full((LANES,), lane, jnp.int32)
    return lax.gather(vec, idx[:, None], _GATHER_DNUMS, (1,),
                      mode=lax.GatherScatterMode.PROMISE_IN_BOUNDS)


def _sc_kernel_body(cd_hbm, ovl_hbm, out_hbm,
                    in0, in1, out0, out1, in_r, out_r, ovl_v,
                    sem_in0, sem_in1, sem_out0, sem_out1, sem_r):
    wid = lax.axis_index("s") * NUM_CORES + lax.axis_index("c")

    first_in = pltpu.make_async_copy(
        cd_hbm.at[:, pl.ds(pl.multiple_of(wid * CW, CW), CW)], in0, sem_in0)
    first_in.start()

    pltpu.sync_copy(ovl_hbm, ovl_v)
    ovl_vecs = [ovl_v[pl.ds(LANES * g, LANES)] for g in range(NON_ZERO // LANES)]

    in_bufs = (in0, in1)
    out_bufs = (out0, out1)
    sem_ins = (sem_in0, sem_in1)
    sem_outs = (sem_out0, sem_out1)

    zeros = jnp.zeros((LANES,), jnp.float32)

    def zero_row(r, carry):
        for ob in out_bufs:
            for v in range(VPC):
                ob[NON_ZERO + r, pl.ds(LANES * v, LANES)] = zeros
        for v in range(VPC_REM):
            out_r[NON_ZERO + r, pl.ds(LANES * v, LANES)] = zeros
        return carry

    lax.fori_loop(0, FEATURES_DIM - NON_ZERO, zero_row, 0)

    def compute_cols(in_ref, out_ref, n_vregs):
        def col_body(v, carry):
            off = pl.ds(LANES * v, LANES)
            for src in range(IN_DIM):
                row = in_ref[src, off]
                for j, s in enumerate(_SRC):
                    if s != src:
                        continue
                    scale = _splat(ovl_vecs[j // LANES], j % LANES)
                    out_ref[j, off] = row * scale
            return carry

        lax.fori_loop(0, n_vregs, col_body, 0)

    @pl.when(wid == REM_WORKER)
    def _():
        rem = pl.ds(REM_OFF, REM)
        cp = pltpu.make_async_copy(cd_hbm.at[:, rem], in_r, sem_r)
        cp.start()
        cp.wait()
        compute_cols(in_r, out_r, VPC_REM)
        cp2 = pltpu.make_async_copy(out_r, out_hbm.at[:, rem], sem_r)
        cp2.start()
        cp2.wait()

    my_chunks = (NUM_FULL - 1 - wid) // NUM_WORKERS + 1

    def col_of(j):
        return pl.multiple_of((wid + j * NUM_WORKERS) * CW, CW)

    def in_copy(j, b):
        return pltpu.make_async_copy(
            cd_hbm.at[:, pl.ds(col_of(j), CW)], in_bufs[b], sem_ins[b])

    def out_copy(j, b):
        return pltpu.make_async_copy(
            out_bufs[b], out_hbm.at[:, pl.ds(col_of(j), CW)], sem_outs[b])

    def pair_body(i, carry):
        for b in range(2):
            j = i * 2 + b

            @pl.when(j < my_chunks)
            def _():
                @pl.when(j + 1 < my_chunks)
                def _():
                    in_copy(j + 1, 1 - b).start()

                in_copy(j, b).wait()

                @pl.when(j >= 2)
                def _():
                    out_copy(j, b).wait()

                compute_cols(in_bufs[b], out_bufs[b], VPC)
                out_copy(j, b).start()
        return carry

    lax.fori_loop(0, MAX_PAIRS, pair_body, 0)

    for b in range(2):
        @pl.when(my_chunks >= b + 1)
        def _():
            out_copy(b, b).wait()


def kernel(charge_density, overlap_constants, select_indices):
    del select_indices
    cd_t = charge_density.T
    mesh = plsc.VectorSubcoreMesh(core_axis_name="c", subcore_axis_name="s")
    run = functools.partial(
        pl.kernel,
        mesh=mesh,
        out_type=jax.ShapeDtypeStruct((FEATURES_DIM, N_NODES), jnp.float32),
        scratch_types=[
            pltpu.VMEM((IN_DIM, CW), jnp.float32),
            pltpu.VMEM((IN_DIM, CW), jnp.float32),
            pltpu.VMEM((FEATURES_DIM, CW), jnp.float32),
            pltpu.VMEM((FEATURES_DIM, CW), jnp.float32),
            pltpu.VMEM((IN_DIM, REM), jnp.float32),
            pltpu.VMEM((FEATURES_DIM, REM), jnp.float32),
            pltpu.VMEM((NON_ZERO,), jnp.float32),
            pltpu.SemaphoreType.DMA,
            pltpu.SemaphoreType.DMA,
            pltpu.SemaphoreType.DMA,
            pltpu.SemaphoreType.DMA,
            pltpu.SemaphoreType.DMA,
        ],
    )(_sc_kernel_body)
    out_t = run(cd_t, overlap_constants)
    return out_t.T


# speedup vs baseline: 1.2402x; 1.0024x over previous
import functools

import jax
import jax.numpy as jnp
from jax import lax
from jax.experimental import pallas as pl
from jax.experimental.pallas import tpu as pltpu
from jax.experimental.pallas import tpu_sc as plsc

N_NODES = 100000
IN_DIM = 16
NON_ZERO = 128
FEATURES_DIM = 200
NUM_CORES = 2
NUM_SUBCORES = 16
NUM_WORKERS = NUM_CORES * NUM_SUBCORES
LANES = 16

CW = 128
NUM_FULL = N_NODES // CW
REM = N_NODES - NUM_FULL * CW
REM_OFF = NUM_FULL * CW
REM_WORKER = 30
VPC = CW // LANES
VPC_REM = REM // LANES
MAX_PAIRS = ((NUM_FULL + NUM_WORKERS - 1) // NUM_WORKERS + 1) // 2

_SRC = [ll * ll + m
        for ll in range(4) for _s in range(8) for m in range(2 * ll + 1)]
assert len(_SRC) == NON_ZERO

_GATHER_DNUMS = lax.GatherDimensionNumbers(
    offset_dims=(), collapsed_slice_dims=(0,), start_index_map=(0,))


def _splat(vec, lane):
    idx = jnp.full((LANES,), lane, jnp.int32)
    return lax.gather(vec, idx[:, None], _GATHER_DNUMS, (1,),
                      mode=lax.GatherScatterMode.PROMISE_IN_BOUNDS)


def _sc_kernel_body(cd_hbm, ovl_hbm, out_hbm,
                    in0, in1, out0, out1, in_r, out_r, ovl_v,
                    sem_in0, sem_in1, sem_out0, sem_out1, sem_r):
    wid = lax.axis_index("s") * NUM_CORES + lax.axis_index("c")

    first_col = pl.multiple_of(wid * NUM_FULL // NUM_WORKERS * CW, CW)
    first_in = pltpu.make_async_copy(
        cd_hbm.at[:, pl.ds(first_col, CW)], in0, sem_in0)
    first_in.start()

    pltpu.sync_copy(ovl_hbm, ovl_v)
    ovl_vecs = [ovl_v[pl.ds(LANES * g, LANES)] for g in range(NON_ZERO // LANES)]

    in_bufs = (in0, in1)
    out_bufs = (out0, out1)
    sem_ins = (sem_in0, sem_in1)
    sem_outs = (sem_out0, sem_out1)

    zeros = jnp.zeros((LANES,), jnp.float32)

    def zero_row(r, carry):
        for ob in out_bufs:
            for v in range(VPC):
                ob[NON_ZERO + r, pl.ds(LANES * v, LANES)] = zeros
        for v in range(VPC_REM):
            out_r[NON_ZERO + r, pl.ds(LANES * v, LANES)] = zeros
        return carry

    lax.fori_loop(0, FEATURES_DIM - NON_ZERO, zero_row, 0)

    def compute_cols(in_ref, out_ref, n_vregs):
        def col_body(v, carry):
            off = pl.ds(LANES * v, LANES)
            for src in range(IN_DIM):
                row = in_ref[src, off]
                for j, s in enumerate(_SRC):
                    if s != src:
                        continue
                    scale = _splat(ovl_vecs[j // LANES], j % LANES)
                    out_ref[j, off] = row * scale
            return carry

        lax.fori_loop(0, n_vregs, col_body, 0)

    @pl.when(wid == REM_WORKER)
    def _():
        rem = pl.ds(REM_OFF, REM)
        cp = pltpu.make_async_copy(cd_hbm.at[:, rem], in_r, sem_r)
        cp.start()
        cp.wait()
        compute_cols(in_r, out_r, VPC_REM)
        cp2 = pltpu.make_async_copy(out_r, out_hbm.at[:, rem], sem_r)
        cp2.start()
        cp2.wait()

    chunk0 = wid * NUM_FULL // NUM_WORKERS
    my_chunks = (wid + 1) * NUM_FULL // NUM_WORKERS - chunk0

    def col_of(j):
        return pl.multiple_of((chunk0 + j) * CW, CW)

    def in_copy(j, b):
        return pltpu.make_async_copy(
            cd_hbm.at[:, pl.ds(col_of(j), CW)], in_bufs[b], sem_ins[b])

    def out_copy(j, b):
        return pltpu.make_async_copy(
            out_bufs[b], out_hbm.at[:, pl.ds(col_of(j), CW)], sem_outs[b])

    def pair_body(i, carry):
        for b in range(2):
            j = i * 2 + b

            @pl.when(j < my_chunks)
            def _():
                @pl.when(j + 1 < my_chunks)
                def _():
                    in_copy(j + 1, 1 - b).start()

                in_copy(j, b).wait()

                @pl.when(j >= 2)
                def _():
                    out_copy(j, b).wait()

                compute_cols(in_bufs[b], out_bufs[b], VPC)
                out_copy(j, b).start()
        return carry

    lax.fori_loop(0, MAX_PAIRS, pair_body, 0)

    for b in range(2):
        @pl.when(my_chunks >= b + 1)
        def _():
            out_copy(b, b).wait()


def kernel(charge_density, overlap_constants, select_indices):
    del select_indices
    cd_t = charge_density.T
    mesh = plsc.VectorSubcoreMesh(core_axis_name="c", subcore_axis_name="s")
    run = functools.partial(
        pl.kernel,
        mesh=mesh,
        out_type=jax.ShapeDtypeStruct((FEATURES_DIM, N_NODES), jnp.float32),
        scratch_types=[
            pltpu.VMEM((IN_DIM, CW), jnp.float32),
            pltpu.VMEM((IN_DIM, CW), jnp.float32),
            pltpu.VMEM((FEATURES_DIM, CW), jnp.float32),
            pltpu.VMEM((FEATURES_DIM, CW), jnp.float32),
            pltpu.VMEM((IN_DIM, REM), jnp.float32),
            pltpu.VMEM((FEATURES_DIM, REM), jnp.float32),
            pltpu.VMEM((NON_ZERO,), jnp.float32),
            pltpu.SemaphoreType.DMA,
            pltpu.SemaphoreType.DMA,
            pltpu.SemaphoreType.DMA,
            pltpu.SemaphoreType.DMA,
            pltpu.SemaphoreType.DMA,
        ],
    )(_sc_kernel_body)
    out_t = run(cd_t, overlap_constants)
    return out_t.T
